# baseline (speedup 1.0000x reference)
import jax
import jax.numpy as jnp
from jax.experimental import pallas as pl
from jax.experimental.pallas import tpu as pltpu
from jax.experimental.pallas import tpu_sc as plsc


BR = 8
LANES = 16


def kernel(x, pos_embed):
    B, N, D = x.shape
    x2 = x.reshape(B * N, D)
    nj = N // BR

    mesh = plsc.VectorSubcoreMesh(core_axis_name="c", subcore_axis_name="s")

    @pl.kernel(
        out_type=jax.ShapeDtypeStruct((B * N, D), x.dtype),
        mesh=mesh,
        scratch_types=[],
    )
    def sc_add(x_hbm, pos_hbm, o_hbm):
        def body(x_vmem, pos_vmem, o_vmem):
            @pl.loop(0, BR)
            def _(r):
                @pl.loop(0, D, step=LANES)
                def _(c):
                    slc = (pl.ds(r, 1), pl.ds(c, LANES))
                    o_vmem.at[*slc][...] = (
                        x_vmem.at[*slc][...] + pos_vmem.at[*slc][...]
                    )

        pltpu.emit_pipeline(
            body,
            grid=(B, nj),
            in_specs=[
                pl.BlockSpec((BR, D), index_map=lambda b, j: (b * nj + j, 0)),
                pl.BlockSpec((BR, D), index_map=lambda b, j: (j, 0)),
            ],
            out_specs=[
                pl.BlockSpec((BR, D), index_map=lambda b, j: (b * nj + j, 0)),
            ],
            core_axis_name=("c", "s"),
            dimension_semantics=(pltpu.PARALLEL, pltpu.PARALLEL),
        )(x_hbm, pos_hbm, o_hbm)

    return sc_add(x2, pos_embed).reshape(B, N, D)


# speedup vs baseline: 4.1501x; 4.1501x over previous
import jax
import jax.numpy as jnp
from jax.experimental import pallas as pl
from jax.experimental.pallas import tpu as pltpu


BN = 2048


def _add_kernel(x_ref, pos_ref, o_ref):
    o_ref[...] = x_ref[...] + pos_ref[...]


def kernel(x, pos_embed):
    B, N, D = x.shape
    grid = (N // BN, B)
    return pl.pallas_call(
        _add_kernel,
        grid=grid,
        in_specs=[
            pl.BlockSpec((1, BN, D), lambda j, b: (b, j, 0)),
            pl.BlockSpec((BN, D), lambda j, b: (j, 0)),
        ],
        out_specs=pl.BlockSpec((1, BN, D), lambda j, b: (b, j, 0)),
        out_shape=jax.ShapeDtypeStruct((B, N, D), x.dtype),
        compiler_params=pltpu.CompilerParams(
            dimension_semantics=("parallel", "parallel")
        ),
    )(x, pos_embed)
